# baseline (speedup 1.0000x reference)
import jax
import jax.numpy as jnp
from jax import lax
from jax.experimental import pallas as pl
from jax.experimental.pallas import tpu as pltpu

_INITIAL_VOCAB = 100000


def _emb_body(idx_smem, emb_in, nemb_in, emb_out, nbuf, tbuf, sem_n, sem_t):
    tok = idx_smem[0]
    exp_row = tok - _INITIAL_VOCAB

    n_in = pltpu.make_async_copy(nemb_in, nbuf, sem_n)
    n_in.start()
    ar = (exp_row // 8) * 8
    e_in = pltpu.make_async_copy(emb_out.at[pl.ds(ar, 8)], tbuf, sem_t)
    e_in.start()

    n_in.wait()
    e_in.wait()
    sub = lax.broadcasted_iota(jnp.int32, (8, 64), 0)
    tbuf[...] = jnp.where(sub == exp_row - ar, nbuf[...], tbuf[...])
    e_out = pltpu.make_async_copy(tbuf, emb_out.at[pl.ds(ar, 8)], sem_t)
    e_out.start()
    e_out.wait()


def _cnt_body(idx_smem, usage_in, ctime_in, nemb_in,
              usage_out, ctime_out, row_out,
              nbuf, ubuf, cbuf, sem_n, sem_u, sem_c):
    tok = idx_smem[0]

    n_in = pltpu.make_async_copy(nemb_in, nbuf, sem_n)
    n_in.start()
    au = (tok // 512) * 512
    u_in = pltpu.make_async_copy(usage_out.at[pl.ds(au, 512)], ubuf, sem_u)
    u_in.start()
    c_in = pltpu.make_async_copy(ctime_out.at[pl.ds(au, 512)], cbuf, sem_c)
    c_in.start()

    n_in.wait()
    row_cp = pltpu.make_async_copy(nbuf.at[pl.ds(0, 2)], row_out, sem_n)
    row_cp.start()

    lane = lax.broadcasted_iota(jnp.int32, (512,), 0)
    u_in.wait()
    ubuf[...] = ubuf[...] + (lane == tok - au).astype(jnp.float32)
    u_out = pltpu.make_async_copy(ubuf, usage_out.at[pl.ds(au, 512)], sem_u)
    u_out.start()

    c_in.wait()
    cbuf[...] = jnp.where(lane == tok - au, jnp.float32(jnp.inf), cbuf[...])
    c_out = pltpu.make_async_copy(cbuf, ctime_out.at[pl.ds(au, 512)], sem_c)
    c_out.start()

    row_cp.wait()
    u_out.wait()
    c_out.wait()


def kernel(token_usage, token_creation_time, expanded_embeddings,
           new_embedding, new_token_id):
    idx = jnp.asarray(new_token_id, jnp.int32).reshape(1)
    n_rows, dim = expanded_embeddings.shape
    nemb8 = jnp.tile(new_embedding, 8).reshape(8, dim)

    expanded = pl.pallas_call(
        _emb_body,
        in_specs=[
            pl.BlockSpec(memory_space=pltpu.SMEM),
            pl.BlockSpec(memory_space=pl.ANY),
            pl.BlockSpec(memory_space=pl.ANY),
        ],
        out_specs=pl.BlockSpec(memory_space=pl.ANY),
        out_shape=jax.ShapeDtypeStruct((n_rows, dim), jnp.float32),
        input_output_aliases={1: 0},
        scratch_shapes=[
            pltpu.VMEM((8, 64), jnp.float32),
            pltpu.VMEM((8, 64), jnp.float32),
            pltpu.SemaphoreType.DMA,
            pltpu.SemaphoreType.DMA,
        ],
    )(idx, expanded_embeddings, nemb8)

    usage, ctime, row = pl.pallas_call(
        _cnt_body,
        in_specs=[
            pl.BlockSpec(memory_space=pltpu.SMEM),
            pl.BlockSpec(memory_space=pl.ANY),
            pl.BlockSpec(memory_space=pl.ANY),
            pl.BlockSpec(memory_space=pl.ANY),
        ],
        out_specs=[
            pl.BlockSpec(memory_space=pl.ANY),
            pl.BlockSpec(memory_space=pl.ANY),
            pl.BlockSpec(memory_space=pl.ANY),
        ],
        out_shape=[
            jax.ShapeDtypeStruct(token_usage.shape, jnp.float32),
            jax.ShapeDtypeStruct(token_creation_time.shape, jnp.float32),
            jax.ShapeDtypeStruct((2, 64), jnp.float32),
        ],
        input_output_aliases={1: 0, 2: 1},
        scratch_shapes=[
            pltpu.VMEM((8, 64), jnp.float32),
            pltpu.VMEM((512,), jnp.float32),
            pltpu.VMEM((512,), jnp.float32),
            pltpu.SemaphoreType.DMA,
            pltpu.SemaphoreType.DMA,
            pltpu.SemaphoreType.DMA,
        ],
    )(idx, token_usage, token_creation_time, nemb8)
    return (row.reshape(-1)[:dim], expanded, usage, ctime)
